# baseline (speedup 1.0000x reference)
import functools

import jax
import jax.numpy as jnp
from jax.experimental import pallas as pl
from jax.experimental.pallas import tpu as pltpu

NUM_EXPERTS = 64
TOP_K = 2
HIDDEN = 2048
T = 16384

ROWS = 2048


def _router_body(h_ref, w_ref, probs_ref, tw_ref, ti_ref):
    logitsT = jax.lax.dot_general(
        w_ref[...], h_ref[...],
        dimension_numbers=(((1,), (1,)), ((), ())),
        preferred_element_type=jnp.float32,
    )
    m = jnp.max(logitsT, axis=0, keepdims=True)
    e = jnp.exp(logitsT - m)
    s = jnp.sum(e, axis=0, keepdims=True)
    pT = e / s
    probs_ref[...] = pT.T

    sub = jax.lax.broadcasted_iota(jnp.int32, pT.shape, 0)
    m1 = jnp.max(pT, axis=0, keepdims=True)
    i1 = jnp.min(jnp.where(pT == m1, sub, NUM_EXPERTS), axis=0, keepdims=True)
    masked = jnp.where(sub == i1, -1.0, pT)
    m2 = jnp.max(masked, axis=0, keepdims=True)
    i2 = jnp.min(jnp.where(masked == m2, sub, NUM_EXPERTS), axis=0, keepdims=True)

    denom = m1 + m2 + 1e-9
    tw = jnp.concatenate([m1 / denom, m2 / denom], axis=0).T
    ti = jnp.concatenate([i1, i2], axis=0).T
    tw_ref[...] = tw.reshape(ROWS // 8, 8, TOP_K)
    ti_ref[...] = ti.reshape(ROWS // 8, 8, TOP_K)


@jax.jit
def _router(hidden_states, weight):
    return pl.pallas_call(
        _router_body,
        grid=(T // ROWS,),
        in_specs=[
            pl.BlockSpec((ROWS, HIDDEN), lambda i: (i, 0)),
            pl.BlockSpec((NUM_EXPERTS, HIDDEN), lambda i: (0, 0)),
        ],
        out_specs=[
            pl.BlockSpec((ROWS, NUM_EXPERTS), lambda i: (i, 0)),
            pl.BlockSpec((ROWS // 8, 8, TOP_K), lambda i: (i, 0, 0)),
            pl.BlockSpec((ROWS // 8, 8, TOP_K), lambda i: (i, 0, 0)),
        ],
        out_shape=[
            jax.ShapeDtypeStruct((T, NUM_EXPERTS), jnp.float32),
            jax.ShapeDtypeStruct((T // 8, 8, TOP_K), jnp.float32),
            jax.ShapeDtypeStruct((T // 8, 8, TOP_K), jnp.int32),
        ],
    )(hidden_states, weight)


def kernel(hidden_states, weight):
    probs, top_w, top_i = _router(hidden_states, weight)
    return (probs,
            top_w.reshape(T, TOP_K).astype(hidden_states.dtype),
            top_i.reshape(T, TOP_K).astype(jnp.int64))


# speedup vs baseline: 1.3055x; 1.3055x over previous
import functools

import jax
import jax.numpy as jnp
from jax.experimental import pallas as pl
from jax.experimental.pallas import tpu as pltpu

NUM_EXPERTS = 64
TOP_K = 2
HIDDEN = 2048
T = 16384

ROWS = 2048


def _router_body(h_ref, w_ref, probs_ref, tw_ref, ti_ref):
    logitsT = jax.lax.dot_general(
        w_ref[...], h_ref[...],
        dimension_numbers=(((1,), (1,)), ((), ())),
        preferred_element_type=jnp.float32,
    )
    m = jnp.max(logitsT, axis=0, keepdims=True)
    e = jnp.exp(logitsT - m)
    s = jnp.sum(e, axis=0, keepdims=True)
    pT = e / s
    probs_ref[...] = pT.T

    sub = jax.lax.broadcasted_iota(jnp.int32, pT.shape, 0)
    m1 = jnp.max(pT, axis=0, keepdims=True)
    i1 = jnp.min(jnp.where(pT == m1, sub, NUM_EXPERTS), axis=0, keepdims=True)
    masked = jnp.where(sub == i1, -1.0, pT)
    m2 = jnp.max(masked, axis=0, keepdims=True)
    i2 = jnp.min(jnp.where(masked == m2, sub, NUM_EXPERTS), axis=0, keepdims=True)

    denom = m1 + m2 + 1e-9
    tw_ref[...] = jnp.concatenate([m1 / denom, m2 / denom], axis=0)
    ti_ref[...] = jnp.concatenate([i1, i2], axis=0)


@jax.jit
def _router(hidden_states, weight):
    return pl.pallas_call(
        _router_body,
        grid=(T // ROWS,),
        in_specs=[
            pl.BlockSpec((ROWS, HIDDEN), lambda i: (i, 0)),
            pl.BlockSpec((NUM_EXPERTS, HIDDEN), lambda i: (0, 0)),
        ],
        out_specs=[
            pl.BlockSpec((ROWS, NUM_EXPERTS), lambda i: (i, 0)),
            pl.BlockSpec((TOP_K, ROWS), lambda i: (0, i)),
            pl.BlockSpec((TOP_K, ROWS), lambda i: (0, i)),
        ],
        out_shape=[
            jax.ShapeDtypeStruct((T, NUM_EXPERTS), jnp.float32),
            jax.ShapeDtypeStruct((TOP_K, T), jnp.float32),
            jax.ShapeDtypeStruct((TOP_K, T), jnp.int32),
        ],
    )(hidden_states, weight)


def kernel(hidden_states, weight):
    probs, top_w, top_i = _router(hidden_states, weight)
    return probs, top_w.T.astype(hidden_states.dtype), top_i.T.astype(jnp.int64)
